# baseline (speedup 1.0000x reference)
import functools

import jax
import jax.numpy as jnp
from jax import lax
from jax.experimental import pallas as pl
from jax.experimental.pallas import tpu as pltpu
from jax.experimental.pallas import tpu_sc as plsc

_L = 16
_NC = 2
_NS = 16
_NW = _NC * _NS
_TOTAL = 16384 * 16
_PER_W = _TOTAL // _NW
_VREGS = _PER_W // _L
_TBL = 128


def _make_lookup():
    mesh = plsc.VectorSubcoreMesh(core_axis_name="c", subcore_axis_name="s")

    @functools.partial(
        pl.kernel,
        mesh=mesh,
        out_type=jax.ShapeDtypeStruct((_TOTAL,), jnp.int32),
        scratch_types=[
            pltpu.VMEM((_PER_W,), jnp.int32),
            pltpu.VMEM((_PER_W,), jnp.int32),
            pltpu.VMEM((_TBL,), jnp.int32),
        ],
        compiler_params=pltpu.CompilerParams(needs_layout_passes=False),
    )
    def _run(x_hbm, table_hbm, out_hbm, x_v, o_v, tab_v):
        wid = lax.axis_index("s") * _NC + lax.axis_index("c")
        base = wid * _PER_W
        pltpu.sync_copy(table_hbm, tab_v)
        pltpu.sync_copy(x_hbm.at[pl.ds(base, _PER_W)], x_v)

        @plsc.parallel_loop(0, _VREGS, unroll=8)
        def _body(i):
            x = x_v[pl.ds(i * _L, _L)]
            y = (x & 0xFFFF) + (x >> 16)
            y = (y & 0xFF) + (y >> 8)
            y = (y & 0xF) + (y >> 4)
            idx = (y << 1) | (x & 1)
            o_v[pl.ds(i * _L, _L)] = plsc.load_gather(tab_v, [idx])

        pltpu.sync_copy(o_v, out_hbm.at[pl.ds(base, _PER_W)])

    return _run


_lookup = _make_lookup()


def kernel(inputs, vocab_values):
    flat = inputs.reshape(-1)
    voc = vocab_values.astype(jnp.int32)
    m5 = jnp.arange(_TBL // 2, dtype=jnp.int32) % 5
    par = m5 & 1
    table = jnp.stack([voc[m5 + 5 * par], voc[m5 + 5 * (par ^ 1)]], axis=1)
    table = table.reshape(_TBL)
    out = _lookup(flat, table)
    return out.reshape(inputs.shape)


# speedup vs baseline: 1.1573x; 1.1573x over previous
import functools

import jax
import jax.numpy as jnp
from jax import lax
from jax.experimental import pallas as pl
from jax.experimental.pallas import tpu as pltpu
from jax.experimental.pallas import tpu_sc as plsc

_L = 16
_NC = 2
_NS = 16
_NW = _NC * _NS
_ROWS = 16384
_COLS = 16
_ROWS_W = _ROWS // _NW
_TBL = 128


def _make_lookup():
    mesh = plsc.VectorSubcoreMesh(core_axis_name="c", subcore_axis_name="s")

    @functools.partial(
        pl.kernel,
        mesh=mesh,
        out_type=jax.ShapeDtypeStruct((_ROWS, _COLS), jnp.int32),
        scratch_types=[
            pltpu.VMEM((_ROWS_W, _COLS), jnp.int32),
            pltpu.VMEM((_TBL,), jnp.int32),
        ],
        compiler_params=pltpu.CompilerParams(needs_layout_passes=False),
    )
    def _run(x_hbm, table_hbm, out_hbm, x_v, tab_v):
        wid = lax.axis_index("s") * _NC + lax.axis_index("c")
        base = wid * _ROWS_W
        pltpu.sync_copy(table_hbm, tab_v)
        pltpu.sync_copy(x_hbm.at[pl.ds(base, _ROWS_W), :], x_v)

        @plsc.parallel_loop(0, _ROWS_W, unroll=8)
        def _body(i):
            x = x_v[i, :]
            y = (x & 0xFFFF) + (x >> 16)
            y = (y & 0xFF) + (y >> 8)
            y = (y & 0xF) + (y >> 4)
            idx = (y << 1) | (x & 1)
            x_v[i, :] = plsc.load_gather(tab_v, [idx])

        pltpu.sync_copy(x_v, out_hbm.at[pl.ds(base, _ROWS_W), :])

    return _run


_lookup = _make_lookup()


def kernel(inputs, vocab_values):
    voc = vocab_values.astype(jnp.int32)
    m5 = jnp.arange(_TBL // 2, dtype=jnp.int32) % 5
    par = m5 & 1
    table = jnp.stack([voc[m5 + 5 * par], voc[m5 + 5 * (par ^ 1)]], axis=1)
    table = table.reshape(_TBL)
    return _lookup(inputs, table)


# speedup vs baseline: 2.0501x; 1.7714x over previous
import functools

import jax
import jax.numpy as jnp
from jax import lax
from jax.experimental import pallas as pl
from jax.experimental.pallas import tpu as pltpu
from jax.experimental.pallas import tpu_sc as plsc

_L = 16
_NC = 2
_NS = 16
_NW = _NC * _NS
_ROWS = 16384
_COLS = 16
_N_W = _ROWS // _NW
_VPC = _N_W // _L
_TBL = 128


def _mod5_small(w):
    return w - 5 * ((w * 205) >> 10)


def _make_lookup():
    mesh = plsc.VectorSubcoreMesh(core_axis_name="c", subcore_axis_name="s")

    @functools.partial(
        pl.kernel,
        mesh=mesh,
        out_type=jax.ShapeDtypeStruct((_COLS, _ROWS), jnp.int32),
        scratch_types=[
            pltpu.VMEM((_COLS, _N_W), jnp.int32),
            pltpu.VMEM((_TBL,), jnp.int32),
            pltpu.VMEM((10,), jnp.int32),
        ],
        compiler_params=pltpu.CompilerParams(needs_layout_passes=False),
    )
    def _run(x_hbm, vocab_hbm, out_hbm, x_v, tab_v, voc_v):
        wid = lax.axis_index("s") * _NC + lax.axis_index("c")
        base = wid * _N_W
        pltpu.sync_copy(vocab_hbm, voc_v)
        pltpu.sync_copy(x_hbm.at[:, pl.ds(base, _N_W)], x_v)

        for t in range(_TBL // _L):
            j = lax.iota(jnp.int32, _L) + t * _L
            y = j >> 1
            b = j & 1
            z = (y & 0xF) + (y >> 4)
            z = (z & 0xF) + (z >> 4)
            m5 = _mod5_small(z)
            m = m5 + 5 * ((m5 & 1) ^ b)
            tab_v[pl.ds(t * _L, _L)] = plsc.load_gather(voc_v, [m])

        for c in range(_COLS):
            @plsc.parallel_loop(0, _VPC, unroll=8)
            def _body(i):
                x = x_v[c, pl.ds(i * _L, _L)]
                y = (x & 0xFFFF) + (x >> 16)
                y = (y & 0xFF) + (y >> 8)
                y = (y & 0xF) + (y >> 4)
                idx = (y << 1) | (x & 1)
                x_v[c, pl.ds(i * _L, _L)] = plsc.load_gather(tab_v, [idx])

        pltpu.sync_copy(x_v, out_hbm.at[:, pl.ds(base, _N_W)])

    return _run


_lookup = _make_lookup()


def kernel(inputs, vocab_values):
    out_t = _lookup(inputs.T, vocab_values.astype(jnp.int32))
    return out_t.T
